# baseline (speedup 1.0000x reference)
import functools

import jax
import jax.numpy as jnp
from jax import lax
from jax.experimental import pallas as pl
from jax.experimental.pallas import tpu as pltpu
from jax.experimental.pallas import tpu_sc as plsc

_MOVIES_NUM = 1000000
_K = 64
_BATCH = 16384
_CHUNK = 128


def _make_kernel():
    info = plsc.get_sparse_core_info()
    nw = info.num_cores * info.num_subcores
    b_per_w = _BATCH // nw
    n_chunks = b_per_w // _CHUNK
    mesh = plsc.VectorSubcoreMesh(core_axis_name="c", subcore_axis_name="s")

    @functools.partial(
        pl.kernel,
        mesh=mesh,
        out_type=(
            jax.ShapeDtypeStruct((_BATCH, _K), jnp.float32),
            jax.ShapeDtypeStruct((_BATCH,), jnp.float32),
        ),
        scratch_types=[
            pltpu.VMEM((n_chunks, _CHUNK), jnp.int32),
            pltpu.VMEM((b_per_w, _K), jnp.float32),
            pltpu.VMEM((b_per_w,), jnp.float32),
            pltpu.SemaphoreType.DMA,
        ],
        compiler_params=pltpu.CompilerParams(use_tc_tiling_on_sc=False),
    )
    def sc_gather(idx_hbm, table_hbm, bias_hbm, emb_out, bias_out,
                  idx_v, rows_v, brows_v, sem):
        wid = lax.axis_index("s") * info.num_cores + lax.axis_index("c")
        base = wid * b_per_w
        pltpu.sync_copy(idx_hbm.at[pl.ds(wid * n_chunks, n_chunks)], idx_v)
        copies = []
        for j in range(n_chunks):
            copies.append(pltpu.async_copy(
                table_hbm.at[idx_v.at[j]],
                rows_v.at[pl.ds(j * _CHUNK, _CHUNK)],
                sem,
            ))
            copies.append(pltpu.async_copy(
                bias_hbm.at[idx_v.at[j]],
                brows_v.at[pl.ds(j * _CHUNK, _CHUNK)],
                sem,
            ))
        for c in copies:
            c.wait()
        pltpu.sync_copy(rows_v, emb_out.at[pl.ds(base, b_per_w)])
        pltpu.sync_copy(brows_v, bias_out.at[pl.ds(base, b_per_w)])

    return sc_gather


_SC_GATHER = _make_kernel()


def kernel(movie_id, movie, bias_movie):
    idx = movie_id.astype(jnp.int32).reshape(_BATCH // _CHUNK, _CHUNK)
    emb, bias = _SC_GATHER(idx, movie, bias_movie.reshape(_MOVIES_NUM))
    return emb, bias.reshape(_BATCH, 1)
